# baseline (speedup 1.0000x reference)
import jax
import jax.numpy as jnp
from jax.experimental import pallas as pl
from jax.experimental.pallas import tpu as pltpu

_BLK = 10000


def _fused_block(obs_ref, x_ref, e_ref, pw1, pb1, pw2, pb2, nw1, nb1, nw2, nb2,
                 out_ref, eout_ref):
    x = x_ref[...]
    obs = obs_ref[...]
    f32 = jnp.float32
    hp = jnp.maximum(jax.lax.dot(x, pw1[...], preferred_element_type=f32) + pb1[...], 0.0)
    yp = jnp.maximum(jax.lax.dot(hp, pw2[...], preferred_element_type=f32) + pb2[...], 0.0)
    hn = jnp.maximum(jax.lax.dot(x, nw1[...], preferred_element_type=f32) + nb1[...], 0.0)
    yn = jnp.maximum(jax.lax.dot(hn, nw2[...], preferred_element_type=f32) + nb2[...], 0.0)
    out_ref[...] = jnp.where(obs == 2, yp, jnp.where(obs == 0, yn, x))
    eout_ref[...] = e_ref[...]


def kernel(node_data, observations, edge_weights, pos_W1, pos_b1, pos_W2, pos_b2,
           neg_W1, neg_b1, neg_W2, neg_b2):
    n, d = node_data.shape
    e = edge_weights.shape[0]
    blk = _BLK
    g = n // blk
    eblk = e // g
    obs = observations.astype(jnp.int32).reshape(n, 1)
    ew = edge_weights.reshape(g, 1, eblk)
    full = lambda i: (0, 0)
    row_blk = lambda i: (i, 0)
    wspec = pl.BlockSpec((d, d), full)
    bspec = pl.BlockSpec((1, d), full)
    out, eout = pl.pallas_call(
        _fused_block,
        grid=(g,),
        in_specs=[
            pl.BlockSpec((blk, 1), row_blk),
            pl.BlockSpec((blk, d), row_blk),
            pl.BlockSpec((1, 1, eblk), lambda i: (i, 0, 0)),
            wspec, bspec, wspec, bspec,
            wspec, bspec, wspec, bspec,
        ],
        out_specs=[
            pl.BlockSpec((blk, d), row_blk),
            pl.BlockSpec((1, 1, eblk), lambda i: (i, 0, 0)),
        ],
        out_shape=[
            jax.ShapeDtypeStruct((n, d), jnp.float32),
            jax.ShapeDtypeStruct((g, 1, eblk), jnp.float32),
        ],
    )(
        obs, node_data, ew,
        pos_W1.T, pos_b1.reshape(1, d), pos_W2.T, pos_b2.reshape(1, d),
        neg_W1.T, neg_b1.reshape(1, d), neg_W2.T, neg_b2.reshape(1, d),
    )
    return out, eout.reshape(e)


# speedup vs baseline: 1.6958x; 1.6958x over previous
import jax
import jax.numpy as jnp
from jax.experimental import pallas as pl
from jax.experimental.pallas import tpu as pltpu

_BLK = 5000


def _fused_block(obs_ref, x_ref, e_ref, pw1, pb1, pw2, pb2, nw1, nb1, nw2, nb2,
                 out_ref, eout_ref):
    x = x_ref[...]
    obs = obs_ref[...]
    f32 = jnp.float32
    bf = jnp.bfloat16
    xb = x.astype(bf)
    hp = jnp.maximum(jax.lax.dot(xb, pw1[...], preferred_element_type=f32) + pb1[...], 0.0)
    yp = jnp.maximum(jax.lax.dot(hp.astype(bf), pw2[...], preferred_element_type=f32) + pb2[...], 0.0)
    hn = jnp.maximum(jax.lax.dot(xb, nw1[...], preferred_element_type=f32) + nb1[...], 0.0)
    yn = jnp.maximum(jax.lax.dot(hn.astype(bf), nw2[...], preferred_element_type=f32) + nb2[...], 0.0)
    out_ref[...] = jnp.where(obs == 2, yp, jnp.where(obs == 0, yn, x))
    eout_ref[...] = e_ref[...]


def kernel(node_data, observations, edge_weights, pos_W1, pos_b1, pos_W2, pos_b2,
           neg_W1, neg_b1, neg_W2, neg_b2):
    n, d = node_data.shape
    e = edge_weights.shape[0]
    blk = _BLK
    g = n // blk
    eblk = e // g
    obs = observations.astype(jnp.int32).reshape(n, 1)
    full = lambda i: (0, 0)
    row_blk = lambda i: (i, 0)
    wspec = pl.BlockSpec((d, d), full)
    bspec = pl.BlockSpec((1, d), full)
    out, eout = pl.pallas_call(
        _fused_block,
        grid=(g,),
        in_specs=[
            pl.BlockSpec((blk, 1), row_blk),
            pl.BlockSpec((blk, d), row_blk),
            pl.BlockSpec((e,), lambda i: (0,)),
            wspec, bspec, wspec, bspec,
            wspec, bspec, wspec, bspec,
        ],
        out_specs=[
            pl.BlockSpec((blk, d), row_blk),
            pl.BlockSpec((e,), lambda i: (0,)),
        ],
        out_shape=[
            jax.ShapeDtypeStruct((n, d), jnp.float32),
            jax.ShapeDtypeStruct((e,), jnp.float32),
        ],
    )(
        obs, node_data, edge_weights,
        pos_W1.T.astype(jnp.bfloat16), pos_b1.reshape(1, d), pos_W2.T.astype(jnp.bfloat16), pos_b2.reshape(1, d),
        neg_W1.T.astype(jnp.bfloat16), neg_b1.reshape(1, d), neg_W2.T.astype(jnp.bfloat16), neg_b2.reshape(1, d),
    )
    return out, eout


# speedup vs baseline: 1.7650x; 1.0408x over previous
import jax
import jax.numpy as jnp
from jax.experimental import pallas as pl
from jax.experimental.pallas import tpu as pltpu

_BLK = 10000


def _fused_block(obs_ref, x_ref, e_ref, pw1, pb1, pw2, pb2, nw1, nb1, nw2, nb2,
                 out_ref, eout_ref):
    x = x_ref[...]
    obs = obs_ref[...]
    f32 = jnp.float32
    bf = jnp.bfloat16
    xb = x.astype(bf)
    hp = jnp.maximum(jax.lax.dot(xb, pw1[...], preferred_element_type=f32) + pb1[...], 0.0)
    yp = jnp.maximum(jax.lax.dot(hp.astype(bf), pw2[...], preferred_element_type=f32) + pb2[...], 0.0)
    hn = jnp.maximum(jax.lax.dot(xb, nw1[...], preferred_element_type=f32) + nb1[...], 0.0)
    yn = jnp.maximum(jax.lax.dot(hn.astype(bf), nw2[...], preferred_element_type=f32) + nb2[...], 0.0)
    out_ref[...] = jnp.where(obs == 2, yp, jnp.where(obs == 0, yn, x))
    eout_ref[...] = e_ref[...]


def kernel(node_data, observations, edge_weights, pos_W1, pos_b1, pos_W2, pos_b2,
           neg_W1, neg_b1, neg_W2, neg_b2):
    n, d = node_data.shape
    e = edge_weights.shape[0]
    blk = _BLK
    g = n // blk
    eblk = e // g
    obs = observations.astype(jnp.int32).reshape(n, 1)
    full = lambda i: (0, 0)
    row_blk = lambda i: (i, 0)
    wspec = pl.BlockSpec((d, d), full)
    bspec = pl.BlockSpec((1, d), full)
    out, eout = pl.pallas_call(
        _fused_block,
        grid=(g,),
        in_specs=[
            pl.BlockSpec((blk, 1), row_blk),
            pl.BlockSpec((blk, d), row_blk),
            pl.BlockSpec((e,), lambda i: (0,)),
            wspec, bspec, wspec, bspec,
            wspec, bspec, wspec, bspec,
        ],
        out_specs=[
            pl.BlockSpec((blk, d), row_blk),
            pl.BlockSpec((e,), lambda i: (0,)),
        ],
        out_shape=[
            jax.ShapeDtypeStruct((n, d), jnp.float32),
            jax.ShapeDtypeStruct((e,), jnp.float32),
        ],
        compiler_params=pltpu.CompilerParams(vmem_limit_bytes=114 * 1024 * 1024),
    )(
        obs, node_data, edge_weights,
        pos_W1.T.astype(jnp.bfloat16), pos_b1.reshape(1, d), pos_W2.T.astype(jnp.bfloat16), pos_b2.reshape(1, d),
        neg_W1.T.astype(jnp.bfloat16), neg_b1.reshape(1, d), neg_W2.T.astype(jnp.bfloat16), neg_b2.reshape(1, d),
    )
    return out, eout
